# baseline (speedup 1.0000x reference)
import functools

import jax
import jax.numpy as jnp
from jax import lax
from jax.experimental import pallas as pl
from jax.experimental.pallas import tpu as pltpu
from jax.experimental.pallas import tpu_sc as plsc

N_SPEAKERS = 10000
EMBED = 512
HIDDEN = 1024
BATCH = 16384

_K = 4
_SLICE = BATCH // _K
_BM = 2048


def _mm_body(a_ref, w_ref, b_ref, o_ref):
    o_ref[...] = (
        lax.dot_general(
            a_ref[...], w_ref[...],
            (((1,), (1,)), ((), ())),
            preferred_element_type=jnp.float32,
        )
        + b_ref[...]
    )


def _mm_body_acc(a_ref, w_ref, b_ref, prev_ref, o_ref):
    del prev_ref
    _mm_body(a_ref, w_ref, b_ref, o_ref)


def _project_slice(emb, w, b2d, slice_idx, prev):
    nsteps = _SLICE // _BM
    base = slice_idx * nsteps
    out_spec = pl.BlockSpec((_BM, HIDDEN), lambda j: (base + j, 0))
    in_specs = [
        pl.BlockSpec((_BM, EMBED), lambda j: (j, 0)),
        pl.BlockSpec((HIDDEN, EMBED), lambda j: (0, 0)),
        pl.BlockSpec((1, HIDDEN), lambda j: (0, 0)),
    ]
    out_shape = jax.ShapeDtypeStruct((BATCH, HIDDEN), jnp.float32)
    if prev is None:
        return pl.pallas_call(
            _mm_body,
            grid=(nsteps,),
            in_specs=in_specs,
            out_specs=out_spec,
            out_shape=out_shape,
        )(emb, w, b2d)
    return pl.pallas_call(
        _mm_body_acc,
        grid=(nsteps,),
        in_specs=in_specs + [pl.BlockSpec(memory_space=pl.ANY)],
        out_specs=out_spec,
        out_shape=out_shape,
        input_output_aliases={3: 0},
    )(emb, w, b2d, prev)


_NC = 2
_NS = 16
_NW = _NC * _NS
_C = 32
_NBUF = 3

_sc_mesh = plsc.VectorSubcoreMesh(core_axis_name="c", subcore_axis_name="s")


def _make_sc_gather(offset):
    b_per_w = _SLICE // _NW
    nch = b_per_w // _C

    @functools.partial(
        pl.kernel,
        mesh=_sc_mesh,
        out_type=jax.ShapeDtypeStruct((_SLICE, EMBED), jnp.float32),
        scratch_types=[
            pltpu.VMEM((b_per_w,), jnp.int32),
            pltpu.VMEM((_C, EMBED), jnp.float32),
            pltpu.VMEM((_C, EMBED), jnp.float32),
            pltpu.VMEM((_C, EMBED), jnp.float32),
            pltpu.SemaphoreType.DMA,
            pltpu.SemaphoreType.DMA,
            pltpu.SemaphoreType.DMA,
            pltpu.SemaphoreType.DMA,
            pltpu.SemaphoreType.DMA,
            pltpu.SemaphoreType.DMA,
        ],
    )
    def gather(ids_hbm, tab_hbm, out_hbm, idx_v,
               buf0, buf1, buf2, sg0, sg1, sg2, ss0, ss1, ss2):
        wid = lax.axis_index("s") * _NC + lax.axis_index("c")
        base = wid * b_per_w
        pltpu.sync_copy(ids_hbm.at[pl.ds(offset + base, b_per_w)], idx_v)

        bufs = (buf0, buf1, buf2)
        sg = (sg0, sg1, sg2)
        ss = (ss0, ss1, ss2)

        def start_gather(c):
            return pltpu.async_copy(
                tab_hbm.at[idx_v.at[pl.ds(c * _C, _C)]], bufs[c % _NBUF], sg[c % _NBUF]
            )

        def start_scatter(c):
            return pltpu.async_copy(
                bufs[c % _NBUF], out_hbm.at[pl.ds(base + c * _C, _C)], ss[c % _NBUF]
            )

        gathers = [None] * nch
        scatters = [None] * nch
        gathers[0] = start_gather(0)
        if nch > 1:
            gathers[1] = start_gather(1)
        for c in range(nch):
            gathers[c].wait()
            scatters[c] = start_scatter(c)
            nxt = c + 2
            if nxt < nch:
                if c >= 1:
                    scatters[c - 1].wait()
                gathers[nxt] = start_gather(nxt)
        for c in range(max(0, nch - 3), nch):
            scatters[c].wait()

    return gather


_sc_gathers = [_make_sc_gather(i * _SLICE) for i in range(_K)]


def kernel(speaker_id, speaker_table, proj_w, proj_b):
    ids = speaker_id.astype(jnp.int32)
    b2d = proj_b.reshape(1, HIDDEN)
    out = None
    for i in range(_K):
        emb = _sc_gathers[i](ids, speaker_table)
        out = _project_slice(emb, proj_w, b2d, i, out)
    return out


# speedup vs baseline: 1.0316x; 1.0316x over previous
import functools

import jax
import jax.numpy as jnp
from jax import lax
from jax.experimental import pallas as pl
from jax.experimental.pallas import tpu as pltpu
from jax.experimental.pallas import tpu_sc as plsc

N_SPEAKERS = 10000
EMBED = 512
HIDDEN = 1024
BATCH = 16384

_K = 2
_SLICE = BATCH // _K
_BM = 2048


def _mm_body(a_ref, w_ref, b_ref, o_ref):
    o_ref[...] = (
        lax.dot_general(
            a_ref[...], w_ref[...],
            (((1,), (1,)), ((), ())),
            preferred_element_type=jnp.float32,
        )
        + b_ref[...]
    )


def _mm_body_acc(a_ref, w_ref, b_ref, prev_ref, o_ref):
    del prev_ref
    _mm_body(a_ref, w_ref, b_ref, o_ref)


def _project_slice(emb, w, b2d, slice_idx, prev):
    nsteps = _SLICE // _BM
    base = slice_idx * nsteps
    out_spec = pl.BlockSpec((_BM, HIDDEN), lambda j: (base + j, 0))
    in_specs = [
        pl.BlockSpec((_BM, EMBED), lambda j: (j, 0)),
        pl.BlockSpec((HIDDEN, EMBED), lambda j: (0, 0)),
        pl.BlockSpec((1, HIDDEN), lambda j: (0, 0)),
    ]
    out_shape = jax.ShapeDtypeStruct((BATCH, HIDDEN), jnp.float32)
    if prev is None:
        return pl.pallas_call(
            _mm_body,
            grid=(nsteps,),
            in_specs=in_specs,
            out_specs=out_spec,
            out_shape=out_shape,
        )(emb, w, b2d)
    return pl.pallas_call(
        _mm_body_acc,
        grid=(nsteps,),
        in_specs=in_specs + [pl.BlockSpec(memory_space=pl.ANY)],
        out_specs=out_spec,
        out_shape=out_shape,
        input_output_aliases={3: 0},
    )(emb, w, b2d, prev)


_NC = 2
_NS = 16
_NW = _NC * _NS
_C = 32
_NBUF = 3

_sc_mesh = plsc.VectorSubcoreMesh(core_axis_name="c", subcore_axis_name="s")


def _make_sc_gather(offset):
    b_per_w = _SLICE // _NW
    nch = b_per_w // _C

    @functools.partial(
        pl.kernel,
        mesh=_sc_mesh,
        out_type=jax.ShapeDtypeStruct((_SLICE, EMBED), jnp.float32),
        scratch_types=[
            pltpu.VMEM((b_per_w,), jnp.int32),
            pltpu.VMEM((_C, EMBED), jnp.float32),
            pltpu.VMEM((_C, EMBED), jnp.float32),
            pltpu.VMEM((_C, EMBED), jnp.float32),
            pltpu.SemaphoreType.DMA,
            pltpu.SemaphoreType.DMA,
            pltpu.SemaphoreType.DMA,
            pltpu.SemaphoreType.DMA,
            pltpu.SemaphoreType.DMA,
            pltpu.SemaphoreType.DMA,
        ],
    )
    def gather(ids_hbm, tab_hbm, out_hbm, idx_v,
               buf0, buf1, buf2, sg0, sg1, sg2, ss0, ss1, ss2):
        wid = lax.axis_index("s") * _NC + lax.axis_index("c")
        base = wid * b_per_w
        pltpu.sync_copy(ids_hbm.at[pl.ds(offset + base, b_per_w)], idx_v)

        bufs = (buf0, buf1, buf2)
        sg = (sg0, sg1, sg2)
        ss = (ss0, ss1, ss2)

        def start_gather(c):
            return pltpu.async_copy(
                tab_hbm.at[idx_v.at[pl.ds(c * _C, _C)]], bufs[c % _NBUF], sg[c % _NBUF]
            )

        def start_scatter(c):
            return pltpu.async_copy(
                bufs[c % _NBUF], out_hbm.at[pl.ds(base + c * _C, _C)], ss[c % _NBUF]
            )

        gathers = [None] * nch
        scatters = [None] * nch
        gathers[0] = start_gather(0)
        if nch > 1:
            gathers[1] = start_gather(1)
        for c in range(nch):
            gathers[c].wait()
            scatters[c] = start_scatter(c)
            nxt = c + 2
            if nxt < nch:
                if c >= 1:
                    scatters[c - 1].wait()
                gathers[nxt] = start_gather(nxt)
        for c in range(max(0, nch - 3), nch):
            scatters[c].wait()

    return gather


_sc_gathers = [_make_sc_gather(i * _SLICE) for i in range(_K)]


def kernel(speaker_id, speaker_table, proj_w, proj_b):
    ids = speaker_id.astype(jnp.int32)
    b2d = proj_b.reshape(1, HIDDEN)
    out = None
    for i in range(_K):
        emb = _sc_gathers[i](ids, speaker_table)
        out = _project_slice(emb, proj_w, b2d, i, out)
    return out
